# baseline (speedup 1.0000x reference)
import jax
import jax.numpy as jnp
from jax.experimental import pallas as pl

H, M = 16, 2048
BLK_R = 1024
N_BLK = M // BLK_R
HH = H // 2
NEG_INF = float("-inf")


def _block_maxes(x_ref, b):
    blk = x_ref[0]
    stripe = x_ref[0, :, pl.ds(b * BLK_R, BLK_R)]
    eye = (
        jax.lax.broadcasted_iota(jnp.int32, (BLK_R, BLK_R), 0)
        == jax.lax.broadcasted_iota(jnp.int32, (BLK_R, BLK_R), 1)
    )
    dia_m = jnp.max(jnp.where(eye, stripe, NEG_INF))
    stripe_off = jnp.max(jnp.where(eye, NEG_INF, stripe))
    colmax = jnp.max(blk, axis=0, keepdims=True)
    in_stripe = (jax.lax.broadcasted_iota(jnp.int32, (1, M), 1) // BLK_R) == b
    off_m = jnp.maximum(jnp.max(jnp.where(in_stripe, NEG_INF, colmax)), stripe_off)
    return dia_m, off_m


def _body(x0_ref, x1_ref, dlo_ref, dhi_ref, olo_ref, ohi_ref):
    b = pl.program_id(1)
    d0, o0 = _block_maxes(x0_ref, b)
    d1, o1 = _block_maxes(x1_ref, b)

    @pl.when(b == 0)
    def _():
        for r in (dlo_ref, dhi_ref, olo_ref, ohi_ref):
            r[...] = jnp.full((1, 1, 128), NEG_INF, jnp.float32)

    dlo_ref[...] = jnp.maximum(dlo_ref[...], d0)
    dhi_ref[...] = jnp.maximum(dhi_ref[...], d1)
    olo_ref[...] = jnp.maximum(olo_ref[...], o0)
    ohi_ref[...] = jnp.maximum(ohi_ref[...], o1)


def kernel(x):
    xs = x.reshape(H, M, M)
    part = pl.BlockSpec((1, 1, 128), lambda h, b: (h, 0, 0))
    part_shape = jax.ShapeDtypeStruct((HH, 1, 128), jnp.float32)
    dlo, dhi, olo, ohi = pl.pallas_call(
        _body,
        grid=(HH, N_BLK),
        in_specs=[
            pl.BlockSpec((1, BLK_R, M), lambda h, b: (h, b, 0)),
            pl.BlockSpec((1, BLK_R, M), lambda h, b: (h + HH, b, 0)),
        ],
        out_specs=[part, part, part, part],
        out_shape=[part_shape, part_shape, part_shape, part_shape],
    )(xs, xs)
    diag = jnp.concatenate([dlo[:, 0, 0], dhi[:, 0, 0]])
    off = jnp.concatenate([olo[:, 0, 0], ohi[:, 0, 0]])
    return jnp.concatenate([diag, off])[None, :]
